# baseline (speedup 1.0000x reference)
import functools

import jax
import jax.numpy as jnp
from jax import lax
from jax.experimental import pallas as pl
from jax.experimental.pallas import tpu as pltpu
from jax.experimental.pallas import tpu_sc as plsc

N_ENT = 100000
N_REL = 1000
D = 64
W = 128
B = 16384
MULT = 20.0

NC = 2
NS = 16
L = 16
NW = NC * NS
BPW = B // NW
CH = 128
NCHUNK = BPW // CH
NG = CH // L

EBLK = 512


def _prep_e_body(et_ref, ett_ref, out_ref):
    out_ref[:, 0:D] = et_ref[...].T
    out_ref[:, D:W] = ett_ref[...].T


def _prep_r_body(rt_ref, rhtt_ref, rttt_ref, rp_ref, rhtt_out_ref):
    r = rt_ref[...].T
    rp_ref[:, 0:D] = r
    rp_ref[:, D:W] = jnp.zeros_like(r)
    rhtt_out_ref[:, 0:D] = rhtt_ref[...].T
    rhtt_out_ref[:, D:W] = rttt_ref[...].T


_prep_e = pl.pallas_call(
    _prep_e_body,
    grid=(pl.cdiv(N_ENT, EBLK),),
    in_specs=[
        pl.BlockSpec((D, EBLK), lambda i: (0, i)),
        pl.BlockSpec((D, EBLK), lambda i: (0, i)),
    ],
    out_specs=pl.BlockSpec((EBLK, W), lambda i: (i, 0)),
    out_shape=jax.ShapeDtypeStruct((N_ENT, W), jnp.float32),
)

_prep_r = pl.pallas_call(
    _prep_r_body,
    out_shape=[
        jax.ShapeDtypeStruct((N_REL, W), jnp.float32),
        jax.ShapeDtypeStruct((N_REL, W), jnp.float32),
    ],
)

_mesh = plsc.VectorSubcoreMesh(core_axis_name="c", subcore_axis_name="s")


@functools.partial(
    pl.kernel,
    out_type=jax.ShapeDtypeStruct((B,), jnp.float32),
    mesh=_mesh,
    compiler_params=pltpu.CompilerParams(
        needs_layout_passes=False, use_tc_tiling_on_sc=True),
    scratch_types=[
        pltpu.VMEM((CH,), jnp.int32),
        pltpu.VMEM((CH,), jnp.int32),
        pltpu.VMEM((CH,), jnp.int32),
        pltpu.VMEM((CH, W), jnp.float32),
        pltpu.VMEM((CH, W), jnp.float32),
        pltpu.VMEM((CH, W), jnp.float32),
        pltpu.VMEM((CH, W), jnp.float32),
        pltpu.VMEM((CH,), jnp.float32),
        pltpu.SemaphoreType.DMA,
    ],
)
def _sc_score(s_hbm, r_hbm, o_hbm, ee_hbm, rp_hbm, rhtt_hbm,
              out_hbm,
              sidx, ridx, oidx, srow, orow, rrow, rtrow, outv, sem):
    wid = lax.axis_index("s") * NC + lax.axis_index("c")

    def chunk_body(c, carry):
        base = pl.multiple_of(wid * BPW + c * CH, CH)
        pltpu.sync_copy(s_hbm.at[pl.ds(base, CH)], sidx)
        pltpu.sync_copy(r_hbm.at[pl.ds(base, CH)], ridx)
        pltpu.sync_copy(o_hbm.at[pl.ds(base, CH)], oidx)
        cps = [
            pltpu.async_copy(ee_hbm.at[sidx], srow, sem),
            pltpu.async_copy(ee_hbm.at[oidx], orow, sem),
            pltpu.async_copy(rp_hbm.at[ridx], rrow, sem),
            pltpu.async_copy(rhtt_hbm.at[ridx], rtrow, sem),
        ]
        for cp in cps:
            cp.wait()

        lane = lax.iota(jnp.int32, 16)
        for g in range(NG):
            tvec = lane + g * L

            def dim_body(d, accs):
                b_acc, h_acc, t_acc = accs
                dv = (lane + d) & 63
                dv2 = dv + 64
                s_e = plsc.load_gather(srow, [tvec, dv])
                s_t = plsc.load_gather(srow, [tvec, dv2])
                o_e = plsc.load_gather(orow, [tvec, dv])
                o_t = plsc.load_gather(orow, [tvec, dv2])
                r_e = plsc.load_gather(rrow, [tvec, dv])
                r_h = plsc.load_gather(rtrow, [tvec, dv])
                r_t = plsc.load_gather(rtrow, [tvec, dv2])
                return (b_acc + s_e * r_e * o_e,
                        h_acc + s_t * r_h,
                        t_acc + o_t * r_t)

            z = jnp.zeros((L,), jnp.float32)
            b_acc, h_acc, t_acc = lax.fori_loop(0, D, dim_body, (z, z, z))
            res = (MULT
                   / (1.0 + jnp.exp(-b_acc))
                   / (1.0 + jnp.exp(-h_acc))
                   / (1.0 + jnp.exp(-t_acc)))
            outv[pl.ds(g * L, L)] = res

        pltpu.sync_copy(outv, out_hbm.at[pl.ds(base, CH)])
        return carry

    lax.fori_loop(0, NCHUNK, chunk_body, 0)


def kernel(s, r, o, E, R, E_t, R_ht, R_tt):
    ee = _prep_e(E.T, E_t.T)
    rp, rhtt = _prep_r(R.T, R_ht.T, R_tt.T)
    return _sc_score(s, r, o, ee, rp, rhtt)
